# baseline (speedup 1.0000x reference)
import functools

import jax
import jax.numpy as jnp
from jax.experimental import pallas as pl
from jax.experimental.pallas import tpu as pltpu


def _hgcn_kernel(adj_ref, x_ref, xb_ref, kg_ref, bg_ref, w1_ref, w2_ref,
                 out_ref, hg1_ref, hg1b_ref, *, bm, nb):
    i = pl.program_id(0)
    j = jnp.where(i < nb, i, i - nb)
    rows = pl.ds(j * bm, bm)

    @pl.when(i < nb)
    def _layer1():
        a = jnp.dot(adj_ref[...].astype(jnp.bfloat16), xb_ref[...],
                    preferred_element_type=jnp.float32)
        t = jnp.maximum(
            jnp.dot(a, w1_ref[...], preferred_element_type=jnp.float32), 0.0)
        x_blk = x_ref[rows, :]
        g = jax.nn.sigmoid(
            jnp.dot(x_blk, kg_ref[...], preferred_element_type=jnp.float32)
            + bg_ref[...])
        h = g * t + (1.0 - g) * x_blk
        hg1_ref[rows, :] = h
        hg1b_ref[rows, :] = h.astype(jnp.bfloat16)

    @pl.when(i >= nb)
    def _layer2():
        a = jnp.dot(adj_ref[...].astype(jnp.bfloat16), hg1b_ref[...],
                    preferred_element_type=jnp.float32)
        t = jnp.maximum(
            jnp.dot(a, w2_ref[...], preferred_element_type=jnp.float32), 0.0)
        h_blk = hg1_ref[rows, :]
        g = jax.nn.sigmoid(
            jnp.dot(h_blk, kg_ref[...], preferred_element_type=jnp.float32)
            + bg_ref[...])
        out_ref[...] = g * t + (1.0 - g) * h_blk


def kernel(x, adj, kernel_gate, bias_gate, Weight_1, Weight_2):
    n, d = x.shape
    bg = bias_gate.reshape(1, d)
    bm = next(b for b in (400, 200, 80, 40, 16, 8, n) if n % b == 0)
    nb = n // bm

    body = functools.partial(_hgcn_kernel, bm=bm, nb=nb)

    full_spec = pl.BlockSpec((n, d), lambda i: (0, 0))
    sq_spec = pl.BlockSpec((d, d), lambda i: (0, 0))
    bias_spec = pl.BlockSpec((1, d), lambda i: (0, 0))
    adj_spec = pl.BlockSpec((bm, n), lambda i: (jnp.where(i < nb, i, i - nb), 0))
    out_spec = pl.BlockSpec((bm, d), lambda i: (jnp.where(i < nb, 0, i - nb), 0))

    return pl.pallas_call(
        body,
        grid=(2 * nb,),
        in_specs=[adj_spec, full_spec, full_spec, sq_spec, bias_spec,
                  sq_spec, sq_spec],
        out_specs=out_spec,
        out_shape=jax.ShapeDtypeStruct((n, d), jnp.float32),
        scratch_shapes=[pltpu.VMEM((n, d), jnp.float32),
                        pltpu.VMEM((n, d), jnp.bfloat16)],
    )(adj, x, x.astype(jnp.bfloat16), kernel_gate, bg, Weight_1, Weight_2)


# speedup vs baseline: 1.0093x; 1.0093x over previous
import functools

import jax
import jax.numpy as jnp
from jax.experimental import pallas as pl
from jax.experimental.pallas import tpu as pltpu


def _hgcn_kernel(adj_ref, x_ref, kg_ref, bg_ref, w1_ref, w2_ref,
                 out_ref, hg1_ref, *, bm, nb):
    i = pl.program_id(0)
    j = jnp.where(i < nb, i, i - nb)
    rows = pl.ds(j * bm, bm)

    @pl.when(i < nb)
    def _layer1():
        a = jnp.dot(adj_ref[...], x_ref[...],
                    preferred_element_type=jnp.float32)
        t = jnp.maximum(
            jnp.dot(a, w1_ref[...], preferred_element_type=jnp.float32), 0.0)
        x_blk = x_ref[rows, :]
        g = jax.nn.sigmoid(
            jnp.dot(x_blk, kg_ref[...], preferred_element_type=jnp.float32)
            + bg_ref[...])
        hg1_ref[rows, :] = g * t + (1.0 - g) * x_blk

    @pl.when(i >= nb)
    def _layer2():
        a = jnp.dot(adj_ref[...], hg1_ref[...],
                    preferred_element_type=jnp.float32)
        t = jnp.maximum(
            jnp.dot(a, w2_ref[...], preferred_element_type=jnp.float32), 0.0)
        h_blk = hg1_ref[rows, :]
        g = jax.nn.sigmoid(
            jnp.dot(h_blk, kg_ref[...], preferred_element_type=jnp.float32)
            + bg_ref[...])
        out_ref[...] = g * t + (1.0 - g) * h_blk


def kernel(x, adj, kernel_gate, bias_gate, Weight_1, Weight_2):
    n, d = x.shape
    bg = bias_gate.reshape(1, d)
    bm = next(b for b in (400, 200, 80, 40, 16, 8, n) if n % b == 0)
    nb = n // bm

    body = functools.partial(_hgcn_kernel, bm=bm, nb=nb)

    full_spec = pl.BlockSpec((n, d), lambda i: (0, 0))
    sq_spec = pl.BlockSpec((d, d), lambda i: (0, 0))
    bias_spec = pl.BlockSpec((1, d), lambda i: (0, 0))
    adj_spec = pl.BlockSpec((bm, n), lambda i: (jnp.where(i < nb, i, i - nb), 0))
    out_spec = pl.BlockSpec((bm, d), lambda i: (jnp.where(i < nb, 0, i - nb), 0))

    return pl.pallas_call(
        body,
        grid=(2 * nb,),
        in_specs=[adj_spec, full_spec, sq_spec, bias_spec, sq_spec, sq_spec],
        out_specs=out_spec,
        out_shape=jax.ShapeDtypeStruct((n, d), jnp.float32),
        scratch_shapes=[pltpu.VMEM((n, d), jnp.float32)],
    )(adj, x, kernel_gate, bg, Weight_1, Weight_2)


# speedup vs baseline: 1.0161x; 1.0067x over previous
import functools

import jax
import jax.numpy as jnp
from jax.experimental import pallas as pl
from jax.experimental.pallas import tpu as pltpu


def _hgcn_kernel(adj_ref, x_ref, kg_ref, bg_ref, w1_ref, w2_ref,
                 out_ref, hg1_ref, xw_ref, *, bm, nb):
    i = pl.program_id(0)
    j = jnp.where(i < nb, i, i - nb)
    rows = pl.ds(j * bm, bm)

    @pl.when(i == 0)
    def _weight_pass_1():
        xw_ref[...] = jnp.dot(x_ref[...], w1_ref[...],
                              preferred_element_type=jnp.float32)

    @pl.when(i == nb)
    def _weight_pass_2():
        xw_ref[...] = jnp.dot(hg1_ref[...], w2_ref[...],
                              preferred_element_type=jnp.float32)

    @pl.when(i < nb)
    def _layer1():
        t = jnp.maximum(
            jnp.dot(adj_ref[...], xw_ref[...],
                    preferred_element_type=jnp.float32), 0.0)
        x_blk = x_ref[rows, :]
        g = jax.nn.sigmoid(
            jnp.dot(x_blk, kg_ref[...], preferred_element_type=jnp.float32)
            + bg_ref[...])
        hg1_ref[rows, :] = g * t + (1.0 - g) * x_blk

    @pl.when(i >= nb)
    def _layer2():
        t = jnp.maximum(
            jnp.dot(adj_ref[...], xw_ref[...],
                    preferred_element_type=jnp.float32), 0.0)
        h_blk = hg1_ref[rows, :]
        g = jax.nn.sigmoid(
            jnp.dot(h_blk, kg_ref[...], preferred_element_type=jnp.float32)
            + bg_ref[...])
        out_ref[...] = g * t + (1.0 - g) * h_blk


def kernel(x, adj, kernel_gate, bias_gate, Weight_1, Weight_2):
    n, d = x.shape
    bg = bias_gate.reshape(1, d)
    bm = next(b for b in (400, 200, 80, 40, 16, 8, n) if n % b == 0)
    nb = n // bm

    body = functools.partial(_hgcn_kernel, bm=bm, nb=nb)

    full_spec = pl.BlockSpec((n, d), lambda i: (0, 0))
    sq_spec = pl.BlockSpec((d, d), lambda i: (0, 0))
    bias_spec = pl.BlockSpec((1, d), lambda i: (0, 0))
    adj_spec = pl.BlockSpec((bm, n), lambda i: (jnp.where(i < nb, i, i - nb), 0))
    out_spec = pl.BlockSpec((bm, d), lambda i: (jnp.where(i < nb, 0, i - nb), 0))

    return pl.pallas_call(
        body,
        grid=(2 * nb,),
        in_specs=[adj_spec, full_spec, sq_spec, bias_spec, sq_spec, sq_spec],
        out_specs=out_spec,
        out_shape=jax.ShapeDtypeStruct((n, d), jnp.float32),
        scratch_shapes=[pltpu.VMEM((n, d), jnp.float32),
                        pltpu.VMEM((n, d), jnp.float32)],
    )(adj, x, kernel_gate, bg, Weight_1, Weight_2)
